# Initial kernel scaffold; baseline (speedup 1.0000x reference)
import functools

import jax
import jax.numpy as jnp
from jax import lax
from jax.experimental import pallas as pl
from jax.experimental.pallas import tpu as pltpu
from jax.experimental.pallas import tpu_sc as plsc

N = 100000
N_SPECIES = 119
L = 16
NC = 2
NS = 16
NW = NC * NS
P = 3136
LAST = N - (NW - 1) * P

_mesh = plsc.VectorSubcoreMesh(core_axis_name="c", subcore_axis_name="s")


@functools.partial(
    pl.kernel,
    mesh=_mesh,
    out_type=jax.ShapeDtypeStruct((N,), jnp.float32),
    scratch_types=[
        pltpu.VMEM((P,), jnp.float32),
        pltpu.VMEM((P,), jnp.int32),
        pltpu.VMEM((P,), jnp.float32),
        pltpu.VMEM((N_SPECIES,), jnp.float32),
    ],
)
def _per_species_offset(x_hbm, z_hbm, off_hbm, out_hbm, x_v, z_v, o_v, tab_v):
    wid = lax.axis_index("s") * NC + lax.axis_index("c")
    base = wid * P
    is_last = wid == NW - 1

    pltpu.sync_copy(off_hbm, tab_v)

    @pl.when(jnp.logical_not(is_last))
    def _():
        pltpu.sync_copy(x_hbm.at[pl.ds(base, P)], x_v)
        pltpu.sync_copy(z_hbm.at[pl.ds(base, P)], z_v)

    @pl.when(is_last)
    def _():
        pltpu.sync_copy(x_hbm.at[pl.ds(base, LAST)], x_v.at[pl.ds(0, LAST)])
        pltpu.sync_copy(z_hbm.at[pl.ds(base, LAST)], z_v.at[pl.ds(0, LAST)])

    n_vec = jnp.where(is_last, LAST // L, P // L)

    def body(i, carry):
        s = i * L
        zz = z_v[pl.ds(s, L)]
        xx = x_v[pl.ds(s, L)]
        o_v[pl.ds(s, L)] = xx + plsc.load_gather(tab_v, [zz])
        return carry

    lax.fori_loop(0, n_vec, body, 0)

    @pl.when(jnp.logical_not(is_last))
    def _():
        pltpu.sync_copy(o_v, out_hbm.at[pl.ds(base, P)])

    @pl.when(is_last)
    def _():
        pltpu.sync_copy(o_v.at[pl.ds(0, LAST)], out_hbm.at[pl.ds(base, LAST)])


def kernel(x, Z, offsets):
    return _per_species_offset(x, Z.astype(jnp.int32), offsets)


# speedup vs baseline: 23.2084x; 23.2084x over previous
import functools

import jax
import jax.numpy as jnp
from jax import lax
from jax.experimental import pallas as pl
from jax.experimental.pallas import tpu as pltpu
from jax.experimental.pallas import tpu_sc as plsc

N = 100000
N_SPECIES = 119
L = 16
NC = 2
NS = 16
NW = NC * NS
P = 3136
LAST = N - (NW - 1) * P

_mesh = plsc.VectorSubcoreMesh(core_axis_name="c", subcore_axis_name="s")


@functools.partial(
    pl.kernel,
    mesh=_mesh,
    out_type=jax.ShapeDtypeStruct((N,), jnp.float32),
    scratch_types=[
        pltpu.VMEM((P,), jnp.float32),
        pltpu.VMEM((P,), jnp.int32),
        pltpu.VMEM((P,), jnp.float32),
        pltpu.VMEM((N_SPECIES,), jnp.float32),
    ],
    compiler_params=pltpu.CompilerParams(needs_layout_passes=False),
)
def _per_species_offset(x_hbm, z_hbm, off_hbm, out_hbm, x_v, z_v, o_v, tab_v):
    wid = lax.axis_index("s") * NC + lax.axis_index("c")
    base = wid * P
    is_last = wid == NW - 1

    pltpu.sync_copy(off_hbm, tab_v)

    @pl.when(jnp.logical_not(is_last))
    def _():
        pltpu.sync_copy(x_hbm.at[pl.ds(base, P)], x_v)
        pltpu.sync_copy(z_hbm.at[pl.ds(base, P)], z_v)

    @pl.when(is_last)
    def _():
        pltpu.sync_copy(x_hbm.at[pl.ds(base, LAST)], x_v.at[pl.ds(0, LAST)])
        pltpu.sync_copy(z_hbm.at[pl.ds(base, LAST)], z_v.at[pl.ds(0, LAST)])

    n_vec = jnp.where(is_last, LAST // L, P // L)

    def body(i, carry):
        s = i * L
        zz = z_v[pl.ds(s, L)]
        xx = x_v[pl.ds(s, L)]
        o_v[pl.ds(s, L)] = xx + plsc.load_gather(tab_v, [zz])
        return carry

    lax.fori_loop(0, n_vec, body, 0)

    @pl.when(jnp.logical_not(is_last))
    def _():
        pltpu.sync_copy(o_v, out_hbm.at[pl.ds(base, P)])

    @pl.when(is_last)
    def _():
        pltpu.sync_copy(o_v.at[pl.ds(0, LAST)], out_hbm.at[pl.ds(base, LAST)])


def kernel(x, Z, offsets):
    return _per_species_offset(x, Z.astype(jnp.int32), offsets)


# speedup vs baseline: 26.1387x; 1.1263x over previous
import functools

import jax
import jax.numpy as jnp
from jax import lax
from jax.experimental import pallas as pl
from jax.experimental.pallas import tpu as pltpu
from jax.experimental.pallas import tpu_sc as plsc

N = 100000
N_SPECIES = 119
L = 16
NC = 2
NS = 16
NW = NC * NS
P = 3136
LAST = N - (NW - 1) * P
LAST_PAD = 2816

_mesh = plsc.VectorSubcoreMesh(core_axis_name="c", subcore_axis_name="s")


@functools.partial(
    pl.kernel,
    mesh=_mesh,
    out_type=jax.ShapeDtypeStruct((N,), jnp.float32),
    scratch_types=[
        pltpu.VMEM((P,), jnp.float32),
        pltpu.VMEM((P,), jnp.int32),
        pltpu.VMEM((P,), jnp.float32),
        pltpu.VMEM((N_SPECIES,), jnp.float32),
        pltpu.SemaphoreType.DMA,
    ],
    compiler_params=pltpu.CompilerParams(needs_layout_passes=False),
)
def _per_species_offset(x_hbm, z_hbm, off_hbm, out_hbm, x_v, z_v, o_v, tab_v,
                        sem):
    wid = lax.axis_index("s") * NC + lax.axis_index("c")
    base = wid * P
    is_last = wid == NW - 1

    tab_cp = pltpu.async_copy(off_hbm, tab_v, sem)

    @pl.when(jnp.logical_not(is_last))
    def _():
        x_cp = pltpu.async_copy(x_hbm.at[pl.ds(base, P)], x_v, sem)
        z_cp = pltpu.async_copy(z_hbm.at[pl.ds(base, P)], z_v, sem)
        x_cp.wait()
        z_cp.wait()

    @pl.when(is_last)
    def _():
        x_cp = pltpu.async_copy(x_hbm.at[pl.ds(base, LAST)],
                                x_v.at[pl.ds(0, LAST)], sem)
        z_cp = pltpu.async_copy(z_hbm.at[pl.ds(base, LAST)],
                                z_v.at[pl.ds(0, LAST)], sem)
        x_cp.wait()
        z_cp.wait()
        zeros = jnp.zeros((L,), jnp.int32)
        z_v[pl.ds(LAST, L)] = zeros
        z_v[pl.ds(LAST + L, L)] = zeros

    tab_cp.wait()

    n_elems = jnp.where(is_last, LAST_PAD, P)

    @plsc.parallel_loop(0, n_elems, L, unroll=4)
    def _(s):
        o_v[pl.ds(s, L)] = x_v[pl.ds(s, L)] + plsc.load_gather(
            tab_v, [z_v[pl.ds(s, L)]])

    @pl.when(jnp.logical_not(is_last))
    def _():
        pltpu.sync_copy(o_v, out_hbm.at[pl.ds(base, P)])

    @pl.when(is_last)
    def _():
        pltpu.sync_copy(o_v.at[pl.ds(0, LAST)], out_hbm.at[pl.ds(base, LAST)])


def kernel(x, Z, offsets):
    return _per_species_offset(x, Z.astype(jnp.int32), offsets)
